# Initial kernel scaffold; baseline (speedup 1.0000x reference)
import jax
import jax.numpy as jnp
from jax.experimental import pallas as pl


def kernel(x, edge_index, W1, b1, W2, b2):
    raise NotImplementedError("write your pallas kernel here")


# speedup vs baseline: 5.6453x; 5.6453x over previous
import functools

import jax
import jax.numpy as jnp
from jax import lax
from jax.experimental import pallas as pl
from jax.experimental.pallas import tpu as pltpu
from jax.experimental.pallas import tpu_sc as plsc

N = 10000
E = 320000
D = 128

NC = 2
NS = 16
NW = NC * NS

NPAD = 10240
ROWS_PER_TILE = NPAD // NS
CH = 128
EPAD = 327680
EW = EPAD // NW
NCHUNKS = EW // CH
TRASH = N + 128

def _deg_body(dst_hbm, ones_hbm, zeros_hbm, out_hbm, idx_v, ones_v, deg_sh):
    c = lax.axis_index("c")
    s = lax.axis_index("s")
    wid = c * NS + s
    r0 = s * ROWS_PER_TILE
    pltpu.sync_copy(zeros_hbm, deg_sh.at[pl.ds(r0, ROWS_PER_TILE)])
    pltpu.sync_copy(ones_hbm, ones_v)
    plsc.subcore_barrier()

    def chunk(t, carry):
        base = wid * EW + t * CH
        pltpu.sync_copy(dst_hbm.at[pl.ds(base, CH)], idx_v)
        pltpu.sync_copy(ones_v, deg_sh.at[idx_v], add=True)
        return carry

    lax.fori_loop(0, NCHUNKS, chunk, 0)
    plsc.subcore_barrier()
    pltpu.sync_copy(deg_sh.at[pl.ds(r0, ROWS_PER_TILE)],
                    out_hbm.at[c, pl.ds(r0, ROWS_PER_TILE)])


def _msg_body(g_hbm, src_hbm, dst_hbm, zeros_hbm, out_hbm,
              sidx_v, didx_v, rows_v, acc_sh, sem):
    c = lax.axis_index("c")
    s = lax.axis_index("s")
    wid = c * NS + s
    r0 = s * ROWS_PER_TILE
    pltpu.sync_copy(zeros_hbm, acc_sh.at[pl.ds(r0, ROWS_PER_TILE)])
    plsc.subcore_barrier()

    def chunk(t, carry):
        base = wid * EW + t * CH
        pltpu.sync_copy(src_hbm.at[pl.ds(base, CH)], sidx_v)
        pltpu.sync_copy(dst_hbm.at[pl.ds(base, CH)], didx_v)
        pltpu.async_copy(g_hbm.at[sidx_v], rows_v, sem).wait()
        pltpu.sync_copy(rows_v, acc_sh.at[didx_v], add=True)
        return carry

    lax.fori_loop(0, NCHUNKS, chunk, 0)
    plsc.subcore_barrier()
    pltpu.sync_copy(acc_sh.at[pl.ds(r0, ROWS_PER_TILE)],
                    out_hbm.at[c, pl.ds(r0, ROWS_PER_TILE)])


def _build_sc(interpret=False):
    mesh = plsc.VectorSubcoreMesh(core_axis_name="c", subcore_axis_name="s",
                                  num_cores=NC, num_subcores=NS)
    deg = pl.kernel(
        _deg_body,
        out_type=jax.ShapeDtypeStruct((NC, NPAD, D), jnp.float32),
        mesh=mesh,
        interpret=interpret,
        scratch_types=[
            pltpu.VMEM((CH,), jnp.int32),
            pltpu.VMEM((CH, D), jnp.float32),
            pltpu.VMEM_SHARED((NPAD, D), jnp.float32),
        ],
    )
    msg = pl.kernel(
        _msg_body,
        out_type=jax.ShapeDtypeStruct((NC, NPAD, D), jnp.float32),
        mesh=mesh,
        interpret=interpret,
        scratch_types=[
            pltpu.VMEM((CH,), jnp.int32),
            pltpu.VMEM((CH,), jnp.int32),
            pltpu.VMEM((CH, D), jnp.float32),
            pltpu.VMEM_SHARED((NPAD, D), jnp.float32),
            pltpu.SemaphoreType.DMA,
        ],
    )
    return deg, msg


_deg_scatter, _msg_scatter = _build_sc()


_RB = 1024
_GRID = NPAD // _RB


def _dinv_of(da_ref, db_ref):
    deg = da_ref[:, 0:1] + db_ref[:, 0:1] + 1.0
    return lax.rsqrt(deg)


def _tc1_body(x_ref, w_ref, da_ref, db_ref, g_ref):
    dinv = _dinv_of(da_ref, db_ref)
    h = jnp.dot(x_ref[...], w_ref[...], preferred_element_type=jnp.float32)
    g_ref[...] = h * dinv


def _tc2_body(g1_ref, s1a_ref, s1b_ref, da_ref, db_ref, b1_ref, w2_ref, g2_ref):
    dinv = _dinv_of(da_ref, db_ref)
    z = dinv * (s1a_ref[...] + s1b_ref[...] + g1_ref[...]) + b1_ref[...]
    z = jnp.maximum(z, 0.0)
    g2_ref[...] = dinv * jnp.dot(z, w2_ref[...],
                                 preferred_element_type=jnp.float32)


def _tc3_body(g2_ref, s2a_ref, s2b_ref, da_ref, db_ref, b2_ref, out_ref):
    dinv = _dinv_of(da_ref, db_ref)
    out_ref[...] = dinv * (s2a_ref[...] + s2b_ref[...] + g2_ref[...]) + b2_ref[...]


_row_spec = pl.BlockSpec((_RB, D), lambda i: (i, 0))
_deg_spec = pl.BlockSpec((_RB, D), lambda i: (i, 0))
_w_spec = pl.BlockSpec((D, D), lambda i: (0, 0))
_b_spec = pl.BlockSpec((1, D), lambda i: (0, 0))
_out_rows = jax.ShapeDtypeStruct((NPAD, D), jnp.float32)

_tc1 = pl.pallas_call(
    _tc1_body, grid=(_GRID,),
    in_specs=[_row_spec, _w_spec, _deg_spec, _deg_spec],
    out_specs=_row_spec, out_shape=_out_rows)

_tc2 = pl.pallas_call(
    _tc2_body, grid=(_GRID,),
    in_specs=[_row_spec, _row_spec, _row_spec, _deg_spec, _deg_spec,
              _b_spec, _w_spec],
    out_specs=_row_spec, out_shape=_out_rows)

_tc3 = pl.pallas_call(
    _tc3_body, grid=(_GRID,),
    in_specs=[_row_spec, _row_spec, _row_spec, _deg_spec, _deg_spec, _b_spec],
    out_specs=_row_spec, out_shape=_out_rows)


def kernel(x, edge_index, W1, b1, W2, b2):
    src = edge_index[0].astype(jnp.int32)
    dst = edge_index[1].astype(jnp.int32)
    srcp = jnp.concatenate([src, jnp.zeros((EPAD - E,), jnp.int32)])
    dstp = jnp.concatenate([dst, jnp.full((EPAD - E,), TRASH, jnp.int32)])
    xpad = jnp.pad(x, ((0, NPAD - N), (0, 0)))

    ones128 = jnp.ones((CH, D), jnp.float32)
    zrows = jnp.zeros((ROWS_PER_TILE, D), jnp.float32)

    deg = _deg_scatter(dstp, ones128, zrows)
    da, db = deg[0], deg[1]

    g1 = _tc1(xpad, W1, da, db)
    s1 = _msg_scatter(g1, srcp, dstp, zrows)
    g2 = _tc2(g1, s1[0], s1[1], da, db, b1.reshape(1, D), W2)
    s2 = _msg_scatter(g2, srcp, dstp, zrows)
    out = _tc3(g2, s2[0], s2[1], da, db, b2.reshape(1, D))
    return out[:N]


# speedup vs baseline: 6.9108x; 1.2242x over previous
import functools

import jax
import jax.numpy as jnp
from jax import lax
from jax.experimental import pallas as pl
from jax.experimental.pallas import tpu as pltpu
from jax.experimental.pallas import tpu_sc as plsc

N = 10000
E = 320000
D = 128

NC = 2
NS = 16
NW = NC * NS

NPAD = 10240
ROWS_PER_TILE = NPAD // NS
CH = 128
EPAD = 327680
EW = EPAD // NW
NCHUNKS = EW // CH
TRASH = N + 128

NBUF = 2
PH = 2
CPP = NCHUNKS // PH


def _deg_body(dst_hbm, ones_hbm, zeros_hbm, out_hbm, didx_all, ones_v, deg_sh):
    c = lax.axis_index("c")
    s = lax.axis_index("s")
    wid = c * NS + s
    r0 = s * ROWS_PER_TILE
    pltpu.sync_copy(zeros_hbm, deg_sh.at[pl.ds(r0, ROWS_PER_TILE)])
    pltpu.sync_copy(ones_hbm, ones_v)
    pltpu.sync_copy(dst_hbm.at[wid], didx_all)
    plsc.subcore_barrier()

    def chunk(t, carry):
        pltpu.sync_copy(ones_v, deg_sh.at[didx_all.at[t]], add=True)
        return carry

    lax.fori_loop(0, NCHUNKS, chunk, 0)
    plsc.subcore_barrier()
    pltpu.sync_copy(deg_sh.at[pl.ds(r0, ROWS_PER_TILE)],
                    out_hbm.at[c, pl.ds(r0, ROWS_PER_TILE)])


def _msg_body(g_hbm, src_hbm, dst_hbm, zeros_hbm, out_hbm,
              sidx_ph, didx_ph, r0_v, r1_v, sem0, sem1, acc_sh):
    c = lax.axis_index("c")
    s = lax.axis_index("s")
    wid = c * NS + s
    row0 = s * ROWS_PER_TILE
    rows = (r0_v, r1_v)
    sems = (sem0, sem1)

    pltpu.sync_copy(zeros_hbm, acc_sh.at[pl.ds(row0, ROWS_PER_TILE)])
    plsc.subcore_barrier()

    def phase(p, carry):
        pltpu.sync_copy(src_hbm.at[wid, pl.ds(p * CPP, CPP)], sidx_ph)
        pltpu.sync_copy(dst_hbm.at[wid, pl.ds(p * CPP, CPP)], didx_ph)
        for b in range(NBUF):
            pltpu.async_copy(g_hbm.at[sidx_ph.at[b]], rows[b], sems[b])

        def outer(o, c2):
            for b in range(NBUF):
                t = o * NBUF + b
                pltpu.make_async_copy(g_hbm.at[sidx_ph.at[t]],
                                      rows[b], sems[b]).wait()
                pltpu.sync_copy(rows[b], acc_sh.at[didx_ph.at[t]], add=True)
                pltpu.async_copy(g_hbm.at[sidx_ph.at[t + NBUF]],
                                 rows[b], sems[b])
            return c2

        lax.fori_loop(0, CPP // NBUF - 1, outer, 0)

        for b in range(NBUF):
            t = CPP - NBUF + b
            pltpu.make_async_copy(g_hbm.at[sidx_ph.at[t]],
                                  rows[b], sems[b]).wait()
            pltpu.sync_copy(rows[b], acc_sh.at[didx_ph.at[t]], add=True)
        return carry

    lax.fori_loop(0, PH, phase, 0)
    plsc.subcore_barrier()
    pltpu.sync_copy(acc_sh.at[pl.ds(row0, ROWS_PER_TILE)],
                    out_hbm.at[c, pl.ds(row0, ROWS_PER_TILE)])


def _build_sc(interpret=False):
    mesh = plsc.VectorSubcoreMesh(core_axis_name="c", subcore_axis_name="s",
                                  num_cores=NC, num_subcores=NS)
    deg = pl.kernel(
        _deg_body,
        out_type=jax.ShapeDtypeStruct((NC, NPAD, D), jnp.float32),
        mesh=mesh,
        interpret=interpret,
        scratch_types=[
            pltpu.VMEM((NCHUNKS, CH), jnp.int32),
            pltpu.VMEM((CH, D), jnp.float32),
            pltpu.VMEM_SHARED((NPAD, D), jnp.float32),
        ],
    )
    msg = pl.kernel(
        _msg_body,
        out_type=jax.ShapeDtypeStruct((NC, NPAD, D), jnp.float32),
        mesh=mesh,
        interpret=interpret,
        scratch_types=[
            pltpu.VMEM((CPP, CH), jnp.int32),
            pltpu.VMEM((CPP, CH), jnp.int32),
            pltpu.VMEM((CH, D), jnp.float32),
            pltpu.VMEM((CH, D), jnp.float32),
            pltpu.SemaphoreType.DMA,
            pltpu.SemaphoreType.DMA,
            pltpu.VMEM_SHARED((NPAD, D), jnp.float32),
        ],
    )
    return deg, msg


_deg_scatter, _msg_scatter = _build_sc()


_RB = 1024
_GRID = NPAD // _RB


def _dinv_of(da_ref, db_ref):
    deg = da_ref[:, 0:1] + db_ref[:, 0:1] + 1.0
    return lax.rsqrt(deg)


def _tc1_body(x_ref, w_ref, da_ref, db_ref, g_ref):
    dinv = _dinv_of(da_ref, db_ref)
    h = jnp.dot(x_ref[...], w_ref[...], preferred_element_type=jnp.float32)
    g_ref[...] = h * dinv


def _tc2_body(g1_ref, s1a_ref, s1b_ref, da_ref, db_ref, b1_ref, w2_ref, g2_ref):
    dinv = _dinv_of(da_ref, db_ref)
    z = dinv * (s1a_ref[...] + s1b_ref[...] + g1_ref[...]) + b1_ref[...]
    z = jnp.maximum(z, 0.0)
    g2_ref[...] = dinv * jnp.dot(z, w2_ref[...],
                                 preferred_element_type=jnp.float32)


def _tc3_body(g2_ref, s2a_ref, s2b_ref, da_ref, db_ref, b2_ref, out_ref):
    dinv = _dinv_of(da_ref, db_ref)
    out_ref[...] = dinv * (s2a_ref[...] + s2b_ref[...] + g2_ref[...]) + b2_ref[...]


_row_spec = pl.BlockSpec((_RB, D), lambda i: (i, 0))
_deg_spec = pl.BlockSpec((_RB, D), lambda i: (i, 0))
_w_spec = pl.BlockSpec((D, D), lambda i: (0, 0))
_b_spec = pl.BlockSpec((1, D), lambda i: (0, 0))
_out_rows = jax.ShapeDtypeStruct((NPAD, D), jnp.float32)

_tc1 = pl.pallas_call(
    _tc1_body, grid=(_GRID,),
    in_specs=[_row_spec, _w_spec, _deg_spec, _deg_spec],
    out_specs=_row_spec, out_shape=_out_rows)

_tc2 = pl.pallas_call(
    _tc2_body, grid=(_GRID,),
    in_specs=[_row_spec, _row_spec, _row_spec, _deg_spec, _deg_spec,
              _b_spec, _w_spec],
    out_specs=_row_spec, out_shape=_out_rows)

_tc3 = pl.pallas_call(
    _tc3_body, grid=(_GRID,),
    in_specs=[_row_spec, _row_spec, _row_spec, _deg_spec, _deg_spec, _b_spec],
    out_specs=_row_spec, out_shape=_out_rows)


def kernel(x, edge_index, W1, b1, W2, b2):
    src = edge_index[0].astype(jnp.int32)
    dst = edge_index[1].astype(jnp.int32)
    srcp = jnp.concatenate([src, jnp.zeros((EPAD - E,), jnp.int32)])
    dstp = jnp.concatenate([dst, jnp.full((EPAD - E,), TRASH, jnp.int32)])
    srcp = srcp.reshape(NW, NCHUNKS, CH)
    dstp = dstp.reshape(NW, NCHUNKS, CH)
    xpad = jnp.pad(x, ((0, NPAD - N), (0, 0)))

    ones128 = jnp.ones((CH, D), jnp.float32)
    zrows = jnp.zeros((ROWS_PER_TILE, D), jnp.float32)

    deg = _deg_scatter(dstp, ones128, zrows)
    da, db = deg[0], deg[1]

    g1 = _tc1(xpad, W1, da, db)
    s1 = _msg_scatter(g1, srcp, dstp, zrows)
    g2 = _tc2(g1, s1[0], s1[1], da, db, b1.reshape(1, D), W2)
    s2 = _msg_scatter(g2, srcp, dstp, zrows)
    out = _tc3(g2, s2[0], s2[1], da, db, b2.reshape(1, D))
    return out[:N]


# speedup vs baseline: 10.6753x; 1.5447x over previous
import functools

import jax
import jax.numpy as jnp
from jax import lax
from jax.experimental import pallas as pl
from jax.experimental.pallas import tpu as pltpu
from jax.experimental.pallas import tpu_sc as plsc

N = 10000
E = 320000
D = 128

NC = 2
NS = 16
NW = NC * NS

NPAD = 10240
ROWS_PER_TILE = NPAD // NS
CH = 128
EPAD = 327680
EW = EPAD // NW
NCHUNKS = EW // CH
TRASH = N + 128

NBUF = 4
DH = D // 2
EW2 = EPAD // NS
NCH2 = EW2 // CH


def _deg_body(dst_hbm, ones_hbm, zeros_hbm, out_hbm, didx_all, ones_v, deg_sh):
    c = lax.axis_index("c")
    s = lax.axis_index("s")
    wid = c * NS + s
    r0 = s * ROWS_PER_TILE
    pltpu.sync_copy(zeros_hbm, deg_sh.at[pl.ds(r0, ROWS_PER_TILE)])
    pltpu.sync_copy(ones_hbm, ones_v)
    pltpu.sync_copy(dst_hbm.at[wid], didx_all)
    plsc.subcore_barrier()

    def chunk(t, carry):
        pltpu.sync_copy(ones_v, deg_sh.at[didx_all.at[t]], add=True)
        return carry

    lax.fori_loop(0, NCHUNKS, chunk, 0)
    plsc.subcore_barrier()
    pltpu.sync_copy(deg_sh.at[pl.ds(r0, ROWS_PER_TILE)],
                    out_hbm.at[c, pl.ds(r0, ROWS_PER_TILE)])


def _msg_body(g_hbm, src_hbm, dst_hbm, zeros_hbm, out_hbm,
              sidx_all, didx_all, r0_v, r1_v, r2_v, r3_v,
              sem0, sem1, sem2, sem3, acc_sh):
    c = lax.axis_index("c")
    s = lax.axis_index("s")
    row0 = s * ROWS_PER_TILE
    rows = (r0_v, r1_v, r2_v, r3_v)
    sems = (sem0, sem1, sem2, sem3)
    gh = g_hbm.at[c]

    pltpu.sync_copy(zeros_hbm, acc_sh.at[pl.ds(row0, ROWS_PER_TILE)])
    pltpu.sync_copy(src_hbm.at[s], sidx_all)
    pltpu.sync_copy(dst_hbm.at[s], didx_all)
    plsc.subcore_barrier()

    for b in range(NBUF):
        pltpu.async_copy(gh.at[sidx_all.at[b]], rows[b], sems[b])

    def outer(o, c2):
        for b in range(NBUF):
            t = o * NBUF + b
            pltpu.make_async_copy(gh.at[sidx_all.at[t]],
                                  rows[b], sems[b]).wait()
            pltpu.sync_copy(rows[b], acc_sh.at[didx_all.at[t]], add=True)
            pltpu.async_copy(gh.at[sidx_all.at[t + NBUF]],
                             rows[b], sems[b])
        return c2

    lax.fori_loop(0, NCH2 // NBUF - 1, outer, 0)

    for b in range(NBUF):
        t = NCH2 - NBUF + b
        pltpu.make_async_copy(gh.at[sidx_all.at[t]],
                              rows[b], sems[b]).wait()
        pltpu.sync_copy(rows[b], acc_sh.at[didx_all.at[t]], add=True)

    plsc.subcore_barrier()
    pltpu.sync_copy(acc_sh.at[pl.ds(row0, ROWS_PER_TILE)],
                    out_hbm.at[c, pl.ds(row0, ROWS_PER_TILE)])


def _build_sc(interpret=False):
    mesh = plsc.VectorSubcoreMesh(core_axis_name="c", subcore_axis_name="s",
                                  num_cores=NC, num_subcores=NS)
    deg = pl.kernel(
        _deg_body,
        out_type=jax.ShapeDtypeStruct((NC, NPAD, D), jnp.float32),
        mesh=mesh,
        interpret=interpret,
        scratch_types=[
            pltpu.VMEM((NCHUNKS, CH), jnp.int32),
            pltpu.VMEM((CH, D), jnp.float32),
            pltpu.VMEM_SHARED((NPAD, D), jnp.float32),
        ],
    )
    msg = pl.kernel(
        _msg_body,
        out_type=jax.ShapeDtypeStruct((NC, NPAD, DH), jnp.float32),
        mesh=mesh,
        compiler_params=pltpu.CompilerParams(use_tc_tiling_on_sc=False),
        interpret=interpret,
        scratch_types=[
            pltpu.VMEM((NCH2, CH), jnp.int32),
            pltpu.VMEM((NCH2, CH), jnp.int32),
            pltpu.VMEM((CH, DH), jnp.float32),
            pltpu.VMEM((CH, DH), jnp.float32),
            pltpu.VMEM((CH, DH), jnp.float32),
            pltpu.VMEM((CH, DH), jnp.float32),
            pltpu.SemaphoreType.DMA,
            pltpu.SemaphoreType.DMA,
            pltpu.SemaphoreType.DMA,
            pltpu.SemaphoreType.DMA,
            pltpu.VMEM_SHARED((NPAD, DH), jnp.float32),
        ],
    )
    return deg, msg


_deg_scatter, _msg_scatter = _build_sc()


_RB = 1024
_GRID = NPAD // _RB


def _dinv_of(da_ref, db_ref):
    deg = da_ref[:, 0:1] + db_ref[:, 0:1] + 1.0
    return lax.rsqrt(deg)


def _tc1_body(x_ref, w_ref, da_ref, db_ref, g_ref):
    dinv = _dinv_of(da_ref, db_ref)
    h = jnp.dot(x_ref[...], w_ref[...], preferred_element_type=jnp.float32)
    g_ref[...] = h * dinv


def _tc2_body(g1_ref, s1_ref, da_ref, db_ref, b1_ref, w2_ref, g2_ref):
    dinv = _dinv_of(da_ref, db_ref)
    z = dinv * (s1_ref[...] + g1_ref[...]) + b1_ref[...]
    z = jnp.maximum(z, 0.0)
    g2_ref[...] = dinv * jnp.dot(z, w2_ref[...],
                                 preferred_element_type=jnp.float32)


def _tc3_body(g2_ref, s2_ref, da_ref, db_ref, b2_ref, out_ref):
    dinv = _dinv_of(da_ref, db_ref)
    out_ref[...] = dinv * (s2_ref[...] + g2_ref[...]) + b2_ref[...]


_row_spec = pl.BlockSpec((_RB, D), lambda i: (i, 0))
_deg_spec = pl.BlockSpec((_RB, D), lambda i: (i, 0))
_w_spec = pl.BlockSpec((D, D), lambda i: (0, 0))
_b_spec = pl.BlockSpec((1, D), lambda i: (0, 0))
_out_rows = jax.ShapeDtypeStruct((NPAD, D), jnp.float32)

_tc1 = pl.pallas_call(
    _tc1_body, grid=(_GRID,),
    in_specs=[_row_spec, _w_spec, _deg_spec, _deg_spec],
    out_specs=_row_spec, out_shape=_out_rows)

_tc2 = pl.pallas_call(
    _tc2_body, grid=(_GRID,),
    in_specs=[_row_spec, _row_spec, _deg_spec, _deg_spec, _b_spec, _w_spec],
    out_specs=_row_spec, out_shape=_out_rows)

_tc3 = pl.pallas_call(
    _tc3_body, grid=(_GRID,),
    in_specs=[_row_spec, _row_spec, _deg_spec, _deg_spec, _b_spec],
    out_specs=_row_spec, out_shape=_out_rows)


def kernel(x, edge_index, W1, b1, W2, b2):
    src = edge_index[0].astype(jnp.int32)
    dst = edge_index[1].astype(jnp.int32)
    srcp = jnp.concatenate([src, jnp.zeros((EPAD - E,), jnp.int32)])
    dstp = jnp.concatenate([dst, jnp.full((EPAD - E,), TRASH, jnp.int32)])
    dstp_deg = dstp.reshape(NW, NCHUNKS, CH)
    srcp2 = srcp.reshape(NS, NCH2, CH)
    dstp2 = dstp.reshape(NS, NCH2, CH)
    xpad = jnp.pad(x, ((0, NPAD - N), (0, 0)))

    ones128 = jnp.ones((CH, D), jnp.float32)
    zrows = jnp.zeros((ROWS_PER_TILE, D), jnp.float32)
    zrowsh = jnp.zeros((ROWS_PER_TILE, DH), jnp.float32)

    def halves(g):
        return g.reshape(NPAD, NC, DH).transpose(1, 0, 2)

    def unhalves(s):
        return s.transpose(1, 0, 2).reshape(NPAD, D)

    deg = _deg_scatter(dstp_deg, ones128, zrows)
    da, db = deg[0], deg[1]

    g1 = _tc1(xpad, W1, da, db)
    s1 = unhalves(_msg_scatter(halves(g1), srcp2, dstp2, zrowsh))
    g2 = _tc2(g1, s1, da, db, b1.reshape(1, D), W2)
    s2 = unhalves(_msg_scatter(halves(g2), srcp2, dstp2, zrowsh))
    out = _tc3(g2, s2, da, db, b2.reshape(1, D))
    return out[:N]


# speedup vs baseline: 11.0982x; 1.0396x over previous
import functools

import jax
import jax.numpy as jnp
from jax import lax
from jax.experimental import pallas as pl
from jax.experimental.pallas import tpu as pltpu
from jax.experimental.pallas import tpu_sc as plsc

N = 10000
E = 320000
D = 128

NC = 2
NS = 16
NW = NC * NS

NPAD = 10240
ROWS_PER_TILE = NPAD // NS
CH = 128
EPAD = 327680
EW = EPAD // NW
NCHUNKS = EW // CH
TRASH = N + 128

NBUF = 4
DH = D // 2
EW2 = EPAD // NS
NCH2 = EW2 // CH


def _deg_body(dst_hbm, ones_hbm, zeros_hbm, out_hbm, didx_all, ones_v, deg_sh):
    c = lax.axis_index("c")
    s = lax.axis_index("s")
    wid = c * NS + s
    r0 = s * ROWS_PER_TILE
    pltpu.sync_copy(zeros_hbm, deg_sh.at[pl.ds(r0, ROWS_PER_TILE)])
    pltpu.sync_copy(ones_hbm, ones_v)
    pltpu.sync_copy(dst_hbm.at[wid], didx_all)
    plsc.subcore_barrier()

    def chunk(t, carry):
        pltpu.sync_copy(ones_v, deg_sh.at[didx_all.at[t]], add=True)
        return carry

    lax.fori_loop(0, NCHUNKS, chunk, 0)
    plsc.subcore_barrier()
    pltpu.sync_copy(deg_sh.at[pl.ds(r0, ROWS_PER_TILE)],
                    out_hbm.at[c, pl.ds(r0, ROWS_PER_TILE)])


def _msg_body(g_hbm, src_hbm, dst_hbm, zeros_hbm, out_hbm,
              sidx_all, didx_all, r0_v, r1_v, r2_v, r3_v,
              sem0, sem1, sem2, sem3, acc_sh):
    c = lax.axis_index("c")
    s = lax.axis_index("s")
    row0 = s * ROWS_PER_TILE
    rows = (r0_v, r1_v, r2_v, r3_v)
    sems = (sem0, sem1, sem2, sem3)
    gh = g_hbm.at[c]

    pltpu.sync_copy(zeros_hbm, acc_sh.at[pl.ds(row0, ROWS_PER_TILE)])
    pltpu.sync_copy(src_hbm.at[s], sidx_all)
    pltpu.sync_copy(dst_hbm.at[s], didx_all)
    plsc.subcore_barrier()

    for b in range(NBUF):
        pltpu.async_copy(gh.at[sidx_all.at[b]], rows[b], sems[b])

    def outer(o, c2):
        for b in range(NBUF):
            t = o * NBUF + b
            pltpu.make_async_copy(gh.at[sidx_all.at[t]],
                                  rows[b], sems[b]).wait()
            pltpu.sync_copy(rows[b], acc_sh.at[didx_all.at[t]], add=True)
            pltpu.async_copy(gh.at[sidx_all.at[t + NBUF]],
                             rows[b], sems[b])
        return c2

    lax.fori_loop(0, NCH2 // NBUF - 1, outer, 0)

    for b in range(NBUF):
        t = NCH2 - NBUF + b
        pltpu.make_async_copy(gh.at[sidx_all.at[t]],
                              rows[b], sems[b]).wait()
        pltpu.sync_copy(rows[b], acc_sh.at[didx_all.at[t]], add=True)

    plsc.subcore_barrier()
    pltpu.sync_copy(acc_sh.at[pl.ds(row0, ROWS_PER_TILE)],
                    out_hbm.at[c, pl.ds(row0, ROWS_PER_TILE)])


def _build_sc(interpret=False):
    mesh = plsc.VectorSubcoreMesh(core_axis_name="c", subcore_axis_name="s",
                                  num_cores=NC, num_subcores=NS)
    deg = pl.kernel(
        _deg_body,
        out_type=jax.ShapeDtypeStruct((NC, NPAD, DH), jnp.float32),
        mesh=mesh,
        compiler_params=pltpu.CompilerParams(use_tc_tiling_on_sc=False),
        interpret=interpret,
        scratch_types=[
            pltpu.VMEM((NCHUNKS, CH), jnp.int32),
            pltpu.VMEM((CH, DH), jnp.float32),
            pltpu.VMEM_SHARED((NPAD, DH), jnp.float32),
        ],
    )
    msg = pl.kernel(
        _msg_body,
        out_type=jax.ShapeDtypeStruct((NC, NPAD, DH), jnp.float32),
        mesh=mesh,
        compiler_params=pltpu.CompilerParams(use_tc_tiling_on_sc=False),
        interpret=interpret,
        scratch_types=[
            pltpu.VMEM((NCH2, CH), jnp.int32),
            pltpu.VMEM((NCH2, CH), jnp.int32),
            pltpu.VMEM((CH, DH), jnp.float32),
            pltpu.VMEM((CH, DH), jnp.float32),
            pltpu.VMEM((CH, DH), jnp.float32),
            pltpu.VMEM((CH, DH), jnp.float32),
            pltpu.SemaphoreType.DMA,
            pltpu.SemaphoreType.DMA,
            pltpu.SemaphoreType.DMA,
            pltpu.SemaphoreType.DMA,
            pltpu.VMEM_SHARED((NPAD, DH), jnp.float32),
        ],
    )
    return deg, msg


_deg_scatter, _msg_scatter = _build_sc()


_RB = 1024
_GRID = NPAD // _RB


def _dinv_of(da_ref, db_ref):
    deg = da_ref[0, :, 0:1] + db_ref[0, :, 0:1] + 1.0
    return lax.rsqrt(deg)


def _mm1_body(x_ref, w_ref, h_ref):
    h_ref[0] = jnp.dot(x_ref[...], w_ref[0],
                       preferred_element_type=jnp.float32)


def _scale1_body(h_ref, da_ref, db_ref, g_ref):
    g_ref[0] = h_ref[0] * _dinv_of(da_ref, db_ref)


def _tc2_body(g1_ref, s1_ref, da_ref, db_ref, b1_ref, w2_ref, g2_ref):
    dinv = _dinv_of(da_ref, db_ref)
    z = jnp.concatenate([dinv * (s1_ref[0] + g1_ref[0]),
                         dinv * (s1_ref[1] + g1_ref[1])], axis=1)
    z = jnp.maximum(z + b1_ref[...], 0.0)
    g2_ref[0] = dinv * jnp.dot(z, w2_ref[0],
                               preferred_element_type=jnp.float32)


def _tc3_body(g2_ref, s2_ref, da_ref, db_ref, b2_ref, out_ref):
    dinv = _dinv_of(da_ref, db_ref)
    out_ref[...] = jnp.concatenate(
        [dinv * (s2_ref[0] + g2_ref[0]),
         dinv * (s2_ref[1] + g2_ref[1])], axis=1) + b2_ref[...]


_half_out = pl.BlockSpec((1, _RB, DH), lambda h, i: (h, i, 0))
_pair_in = pl.BlockSpec((NC, _RB, DH), lambda h, i: (0, i, 0))
_dega_spec = pl.BlockSpec((1, _RB, DH), lambda h, i: (0, i, 0))
_degb_spec = pl.BlockSpec((1, _RB, DH), lambda h, i: (1, i, 0))
_x_spec = pl.BlockSpec((_RB, D), lambda h, i: (i, 0))
_wh_spec = pl.BlockSpec((1, D, DH), lambda h, i: (h, 0, 0))
_b_spec = pl.BlockSpec((1, D), lambda h, i: (0, 0))
_halves_shape = jax.ShapeDtypeStruct((NC, NPAD, DH), jnp.float32)

_mm1 = pl.pallas_call(
    _mm1_body, grid=(NC, _GRID),
    in_specs=[_x_spec, _wh_spec],
    out_specs=_half_out, out_shape=_halves_shape)

_scale1 = pl.pallas_call(
    _scale1_body, grid=(NC, _GRID),
    in_specs=[_half_out, _dega_spec, _degb_spec],
    out_specs=_half_out, out_shape=_halves_shape)

_tc2 = pl.pallas_call(
    _tc2_body, grid=(NC, _GRID),
    in_specs=[_pair_in, _pair_in, _dega_spec, _degb_spec, _b_spec, _wh_spec],
    out_specs=_half_out, out_shape=_halves_shape)

_tc3 = pl.pallas_call(
    _tc3_body, grid=(_GRID,),
    in_specs=[pl.BlockSpec((NC, _RB, DH), lambda i: (0, i, 0)),
              pl.BlockSpec((NC, _RB, DH), lambda i: (0, i, 0)),
              pl.BlockSpec((1, _RB, DH), lambda i: (0, i, 0)),
              pl.BlockSpec((1, _RB, DH), lambda i: (1, i, 0)),
              pl.BlockSpec((1, D), lambda i: (0, 0))],
    out_specs=pl.BlockSpec((_RB, D), lambda i: (i, 0)),
    out_shape=jax.ShapeDtypeStruct((NPAD, D), jnp.float32))


def kernel(x, edge_index, W1, b1, W2, b2):
    src = edge_index[0].astype(jnp.int32)
    dst = edge_index[1].astype(jnp.int32)
    srcp = jnp.concatenate([src, jnp.zeros((EPAD - E,), jnp.int32)])
    dstp = jnp.concatenate([dst, jnp.full((EPAD - E,), TRASH, jnp.int32)])
    dstp_deg = dstp.reshape(NW, NCHUNKS, CH)
    srcp2 = srcp.reshape(NS, NCH2, CH)
    dstp2 = dstp.reshape(NS, NCH2, CH)
    xpad = jnp.pad(x, ((0, NPAD - N), (0, 0)))

    ones64 = jnp.ones((CH, DH), jnp.float32)
    zrowsh = jnp.zeros((ROWS_PER_TILE, DH), jnp.float32)

    W1h = W1.reshape(D, NC, DH).transpose(1, 0, 2)
    W2h = W2.reshape(D, NC, DH).transpose(1, 0, 2)

    deg = _deg_scatter(dstp_deg, ones64, zrowsh)

    h1 = _mm1(xpad, W1h)
    g1 = _scale1(h1, deg, deg)
    s1 = _msg_scatter(g1, srcp2, dstp2, zrowsh)
    g2 = _tc2(g1, s1, deg, deg, b1.reshape(1, D), W2h)
    s2 = _msg_scatter(g2, srcp2, dstp2, zrowsh)
    out = _tc3(g2, s2, deg, deg, b2.reshape(1, D))
    return out[:N]
